# Initial kernel scaffold; baseline (speedup 1.0000x reference)
import jax
import jax.numpy as jnp
from jax.experimental import pallas as pl


def kernel(input_tensor, seq_lengths, shift_step, pos_table):
    raise NotImplementedError("write your pallas kernel here")


# speedup vs baseline: 1.5115x; 1.5115x over previous
import functools

import jax
import jax.numpy as jnp
from jax import lax
from jax.experimental import pallas as pl
from jax.experimental.pallas import tpu as pltpu
from jax.experimental.pallas import tpu_sc as plsc

B = 256
L1 = B * (B - 1) // 2
D = 1024
MAX_POS = 4096
NPAD = 32768

_NC, _NS = 2, 16
_NW = _NC * _NS

_K = 32
_NCH = L1 // _K
_ITERS = -(-_NCH // _NW)


_RB = 4096


def _posid_body(shift_ref, seqr_ref, seqc_ref, out_ref):
    b = pl.program_id(0)
    seq_row = seqr_ref[...].astype(jnp.float32)
    seq_col = seqc_ref[...].astype(jnp.float32)
    r = lax.broadcasted_iota(jnp.int32, (B, B), 0)
    c = lax.broadcasted_iota(jnp.int32, (B, B), 1)
    triu = (r <= c).astype(jnp.float32)
    cs_row = jnp.dot(seq_row, triu, preferred_element_type=jnp.float32)
    ii = b * _RB + lax.broadcasted_iota(jnp.int32, (_RB, 1), 0)
    iif = ii.astype(jnp.float32)
    m2 = (cs_row <= iif).astype(jnp.float32)
    start = jnp.dot(m2, seq_col, preferred_element_type=jnp.float32)
    p = ii + shift_ref[0, 0] - start.astype(jnp.int32)
    out_ref[...] = jnp.clip(p, 0, MAX_POS - 1)


_posid_call = pl.pallas_call(
    _posid_body,
    grid=(NPAD // _RB,),
    in_specs=[
        pl.BlockSpec((1, 1), lambda b: (0, 0), memory_space=pltpu.SMEM),
        pl.BlockSpec((1, B), lambda b: (0, 0)),
        pl.BlockSpec((B, 1), lambda b: (0, 0)),
    ],
    out_specs=pl.BlockSpec((_RB, 1), lambda b: (b, 0)),
    out_shape=jax.ShapeDtypeStruct((NPAD, 1), jnp.int32),
)


def _sc_body(in_hbm, idx_hbm, tab_hbm, out_hbm,
             idx_v, in_v, tab_v, sem_in, sem_tab, sem_out):
    wid = lax.axis_index("s") * _NC + lax.axis_index("c")

    def chunk(k, carry):
        cid = wid + k * _NW

        @pl.when(cid < _NCH)
        def _():
            base = cid * _K
            pltpu.sync_copy(idx_hbm.at[pl.ds(base, _K)], idx_v)
            cin = pltpu.async_copy(in_hbm.at[pl.ds(base, _K)], in_v, sem_in)
            ctab = pltpu.async_copy(tab_hbm.at[idx_v], tab_v, sem_tab)
            cin.wait()
            ctab.wait()

            def row(r, cr):
                for u in range(0, D, 16):
                    in_v[r, pl.ds(u, 16)] = (
                        in_v[r, pl.ds(u, 16)] + tab_v[r, pl.ds(u, 16)]
                    )
                return cr

            lax.fori_loop(0, _K, row, 0)
            pltpu.async_copy(in_v, out_hbm.at[pl.ds(base, _K)], sem_out).wait()

        return carry

    lax.fori_loop(0, _ITERS, chunk, 0)


@functools.cache
def _get_sc_kernel():
    return pl.kernel(
        _sc_body,
        out_type=jax.ShapeDtypeStruct((L1, D), jnp.float32),
        mesh=plsc.VectorSubcoreMesh(core_axis_name="c", subcore_axis_name="s",
                                    num_cores=_NC, num_subcores=_NS),
        scratch_types=[
            pltpu.VMEM((_K,), jnp.int32),
            pltpu.VMEM((_K, D), jnp.float32),
            pltpu.VMEM((_K, D), jnp.float32),
            pltpu.SemaphoreType.DMA,
            pltpu.SemaphoreType.DMA,
            pltpu.SemaphoreType.DMA,
        ],
    )


def kernel(input_tensor, seq_lengths, shift_step, pos_table):
    shift = jnp.asarray(shift_step, jnp.int32).reshape(1, 1)
    seq = jnp.asarray(seq_lengths, jnp.int32)
    pos2d = _posid_call(shift, seq.reshape(1, B), seq.reshape(B, 1))
    pos_flat = pos2d.reshape(NPAD)
    return _get_sc_kernel()(input_tensor, pos_flat, pos_table)


# speedup vs baseline: 2.2468x; 1.4864x over previous
import functools

import jax
import jax.numpy as jnp
from jax import lax
from jax.experimental import pallas as pl
from jax.experimental.pallas import tpu as pltpu
from jax.experimental.pallas import tpu_sc as plsc

B = 256
L1 = B * (B - 1) // 2
D = 1024
MAX_POS = 4096
NPAD = 32768

_NC, _NS = 2, 16
_NW = _NC * _NS

_K = 16
_NCH = L1 // _K
_KMAX = NPAD // (_NW * _K)
_GITERS = 22

_RB = 4096


def _posid_body(shift_ref, seqr_ref, seqc_ref, out_ref):
    blk = pl.program_id(0)
    seq_row = seqr_ref[...].astype(jnp.float32)
    seq_col = seqc_ref[...].astype(jnp.float32)
    r = lax.broadcasted_iota(jnp.int32, (B, B), 0)
    c = lax.broadcasted_iota(jnp.int32, (B, B), 1)
    triu = (r <= c).astype(jnp.float32)
    cs_row = jnp.dot(seq_row, triu, preferred_element_type=jnp.float32)
    j = blk * _RB + lax.broadcasted_iota(jnp.int32, (_RB, 1), 0)
    w = j >> 10
    kk = (j & 1023) >> 4
    rr = j & 15
    ii = (w + kk * _NW) * _K + rr
    iif = ii.astype(jnp.float32)
    m2 = (cs_row <= iif).astype(jnp.float32)
    start = jnp.dot(m2, seq_col, preferred_element_type=jnp.float32)
    p = ii + shift_ref[0, 0] - start.astype(jnp.int32)
    out_ref[...] = jnp.clip(p, 0, MAX_POS - 1)


_posid_call = pl.pallas_call(
    _posid_body,
    grid=(NPAD // _RB,),
    in_specs=[
        pl.BlockSpec((1, 1), lambda b: (0, 0), memory_space=pltpu.SMEM),
        pl.BlockSpec((1, B), lambda b: (0, 0)),
        pl.BlockSpec((B, 1), lambda b: (0, 0)),
    ],
    out_specs=pl.BlockSpec((_RB, 1), lambda b: (b, 0)),
    out_shape=jax.ShapeDtypeStruct((NPAD, 1), jnp.int32),
)


def _sc_body(in_hbm, idx_hbm, tab_hbm, out_hbm, idx_all,
             in0, in1, in2, tab0, tab1, tab2,
             si0, si1, si2, st0, st1, st2, so0, so1, so2):
    in_v = (in0, in1, in2)
    tab_v = (tab0, tab1, tab2)
    sin = (si0, si1, si2)
    stab = (st0, st1, st2)
    sout = (so0, so1, so2)
    wid = lax.axis_index("s") * _NC + lax.axis_index("c")
    pltpu.sync_copy(idx_hbm.at[wid], idx_all)

    def in_copy(cid, b):
        return pltpu.make_async_copy(
            in_hbm.at[pl.ds(cid * _K, _K)], in_v[b], sin[b])

    def tab_copy(k, b):
        return pltpu.make_async_copy(
            tab_hbm.at[idx_all.at[k]], tab_v[b], stab[b])

    def out_copy(cid, b):
        return pltpu.make_async_copy(
            in_v[b], out_hbm.at[pl.ds(cid * _K, _K)], sout[b])

    def issue(k, b):
        cid = wid + k * _NW

        @pl.when(cid < _NCH)
        def _():
            in_copy(cid, b).start()
            tab_copy(k, b).start()

    def outer(g, carry):
        for b3 in range(3):
            k = g * 3 + b3
            b = b3
            bn = (b3 + 1) % 3
            cid_prev = wid + (k - 2) * _NW

            @pl.when(jnp.logical_and(k >= 2, cid_prev < _NCH))
            def _():
                out_copy(cid_prev, bn).wait()

            issue(k + 1, bn)

            cid = wid + k * _NW

            @pl.when(cid < _NCH)
            def _():
                in_copy(cid, b).wait()
                tab_copy(k, b).wait()

                def row(rr, cr):
                    for u in range(0, D, 16):
                        in_v[b][rr, pl.ds(u, 16)] = (
                            in_v[b][rr, pl.ds(u, 16)]
                            + tab_v[b][rr, pl.ds(u, 16)]
                        )
                    return cr

                lax.fori_loop(0, _K, row, 0)
                out_copy(cid, b).start()

        return carry

    issue(0, 0)
    lax.fori_loop(0, _GITERS, outer, 0)


@functools.cache
def _get_sc_kernel():
    return pl.kernel(
        _sc_body,
        out_type=jax.ShapeDtypeStruct((L1, D), jnp.float32),
        mesh=plsc.VectorSubcoreMesh(core_axis_name="c", subcore_axis_name="s",
                                    num_cores=_NC, num_subcores=_NS),
        scratch_types=[
            pltpu.VMEM((_KMAX, _K), jnp.int32),
            pltpu.VMEM((_K, D), jnp.float32),
            pltpu.VMEM((_K, D), jnp.float32),
            pltpu.VMEM((_K, D), jnp.float32),
            pltpu.VMEM((_K, D), jnp.float32),
            pltpu.VMEM((_K, D), jnp.float32),
            pltpu.VMEM((_K, D), jnp.float32),
            pltpu.SemaphoreType.DMA,
            pltpu.SemaphoreType.DMA,
            pltpu.SemaphoreType.DMA,
            pltpu.SemaphoreType.DMA,
            pltpu.SemaphoreType.DMA,
            pltpu.SemaphoreType.DMA,
            pltpu.SemaphoreType.DMA,
            pltpu.SemaphoreType.DMA,
            pltpu.SemaphoreType.DMA,
        ],
    )


def kernel(input_tensor, seq_lengths, shift_step, pos_table):
    shift = jnp.asarray(shift_step, jnp.int32).reshape(1, 1)
    seq = jnp.asarray(seq_lengths, jnp.int32)
    pos2d = _posid_call(shift, seq.reshape(1, B), seq.reshape(B, 1))
    pos_perm = pos2d.reshape(_NW, _KMAX, _K)
    return _get_sc_kernel()(input_tensor, pos_perm, pos_table)
